# baseline (speedup 1.0000x reference)
import functools

import jax
import jax.numpy as jnp
from jax import lax
from jax.experimental import pallas as pl
from jax.experimental.pallas import tpu as pltpu
from jax.experimental.pallas import tpu_sc as plsc

HIDDEN = 128
MAX_SENT = 100
BATCH = 4096
NUM_ELEM = 100
EPS = 1e-5

TILE_N = 8
POS_ROWS = TILE_N * NUM_ELEM
R = 12800
SUB = POS_ROWS


def _sc_gather(ids_hbm, table_hbm, out_hbm, idx_v, rows_v, sem):
    wid = lax.axis_index("s") * 2 + lax.axis_index("c")

    @pl.when(wid == 0)
    def _():
        pltpu.sync_copy(ids_hbm, idx_v)
        descs = [
            pltpu.async_copy(
                table_hbm.at[idx_v],
                rows_v.at[pl.ds(k * NUM_ELEM, NUM_ELEM)], sem)
            for k in range(TILE_N)
        ]
        for d in descs:
            d.wait()
        pltpu.sync_copy(rows_v, out_hbm)


_sc_gather_call = functools.partial(
    pl.kernel,
    out_type=jax.ShapeDtypeStruct((POS_ROWS, HIDDEN), jnp.float32),
    mesh=plsc.VectorSubcoreMesh(core_axis_name="c", subcore_axis_name="s"),
    scratch_types=[
        pltpu.VMEM((NUM_ELEM,), jnp.int32),
        pltpu.VMEM((POS_ROWS, HIDDEN), jnp.float32),
        pltpu.SemaphoreType.DMA,
    ],
)(_sc_gather)


def _tc_body(x_ref, pos_ref, gamma_ref, beta_ref, o_ref):
    jmat = jnp.full((HIDDEN, HIDDEN), 1.0 / HIDDEN, dtype=jnp.float32)
    gamma = gamma_ref[0, :]
    beta = beta_ref[0, :]

    def step(k, _):
        xc = x_ref[pl.ds(k * SUB, SUB), :]
        out = xc + pos_ref[...]
        m = jnp.dot(out, jmat, preferred_element_type=jnp.float32)
        m2 = jnp.dot(out * out, jmat, preferred_element_type=jnp.float32)
        var = m2 - m * m
        res = (out - m) * lax.rsqrt(var + EPS)
        o_ref[pl.ds(k * SUB, SUB), :] = res * gamma + beta
        return 0

    lax.fori_loop(0, R // SUB, step, 0)


@jax.jit
def kernel(batch_elem_emb, sent_pos_ids, emb_table, gamma, beta):
    ids = sent_pos_ids.astype(jnp.int32)
    pos_tiled = _sc_gather_call(ids, emb_table)
    x2 = batch_elem_emb.reshape(BATCH * NUM_ELEM, HIDDEN)
    gamma2 = gamma.reshape(1, HIDDEN)
    beta2 = beta.reshape(1, HIDDEN)
    y = pl.pallas_call(
        _tc_body,
        grid=(BATCH * NUM_ELEM // R,),
        in_specs=[
            pl.BlockSpec((R, HIDDEN), lambda i: (i, 0)),
            pl.BlockSpec((POS_ROWS, HIDDEN), lambda i: (0, 0)),
            pl.BlockSpec((1, HIDDEN), lambda i: (0, 0)),
            pl.BlockSpec((1, HIDDEN), lambda i: (0, 0)),
        ],
        out_specs=pl.BlockSpec((R, HIDDEN), lambda i: (i, 0)),
        out_shape=jax.ShapeDtypeStruct((BATCH * NUM_ELEM, HIDDEN), jnp.float32),
    )(x2, pos_tiled, gamma2, beta2)
    return y.reshape(BATCH, NUM_ELEM, HIDDEN)


# speedup vs baseline: 1.8736x; 1.8736x over previous
import jax
import jax.numpy as jnp
from jax import lax
from jax.experimental import pallas as pl
from jax.experimental.pallas import tpu as pltpu

HIDDEN = 128
MAX_SENT = 100
BATCH = 4096
NUM_ELEM = 100
EPS = 1e-5

C = 64
K = 4
N = BATCH // C
CH = 8


def _body(x_hbm, ids_ref, table_ref, gamma_ref, beta_ref, o_hbm,
          in_buf, out_buf, in_sem, out_sem):
    ids = ids_ref[0, :]
    iota = lax.broadcasted_iota(jnp.int32, (NUM_ELEM, MAX_SENT), 1)
    onehot = (ids[:, None] == iota).astype(jnp.float32)
    pos = jnp.dot(onehot, table_ref[...], preferred_element_type=jnp.float32)
    gamma = gamma_ref[0, :]
    beta = beta_ref[0, :]

    def in_copy(i, slot):
        return pltpu.make_async_copy(
            x_hbm.at[pl.ds(i * C, C)], in_buf.at[slot], in_sem.at[slot])

    def out_copy(i, slot):
        return pltpu.make_async_copy(
            out_buf.at[slot], o_hbm.at[pl.ds(i * C, C)], out_sem.at[slot])

    for s in range(K):
        in_copy(s, s).start(priority=0)

    def iter_fn(i, _):
        slot = lax.rem(i, K)
        in_copy(i, slot).wait()

        @pl.when(i >= K)
        def _wait_out():
            out_copy(i - K, slot).wait()

        def sub(k, _):
            x = in_buf[slot, pl.ds(k * CH, CH)]
            out = x + pos[None, :, :]
            mean = jnp.mean(out, axis=-1, keepdims=True)
            c = out - mean
            var = jnp.mean(c * c, axis=-1, keepdims=True)
            normed = c * lax.rsqrt(var + EPS)
            out_buf[slot, pl.ds(k * CH, CH)] = normed * gamma + beta
            return 0

        lax.fori_loop(0, C // CH, sub, 0)
        out_copy(i, slot).start(priority=1)

        @pl.when(i + K < N)
        def _next_in():
            in_copy(i + K, slot).start(priority=0)

        return 0

    lax.fori_loop(0, N, iter_fn, 0)

    def drain(j, _):
        out_copy(j, lax.rem(j, K)).wait()
        return 0

    lax.fori_loop(N - K, N, drain, 0)


@jax.jit
def kernel(batch_elem_emb, sent_pos_ids, emb_table, gamma, beta):
    ids2 = sent_pos_ids.astype(jnp.int32).reshape(1, NUM_ELEM)
    gamma2 = gamma.reshape(1, HIDDEN)
    beta2 = beta.reshape(1, HIDDEN)
    vm = pltpu.MemorySpace.VMEM
    return pl.pallas_call(
        _body,
        in_specs=[
            pl.BlockSpec(memory_space=pltpu.MemorySpace.HBM),
            pl.BlockSpec(memory_space=vm),
            pl.BlockSpec(memory_space=vm),
            pl.BlockSpec(memory_space=vm),
            pl.BlockSpec(memory_space=vm),
        ],
        out_specs=pl.BlockSpec(memory_space=pltpu.MemorySpace.HBM),
        out_shape=jax.ShapeDtypeStruct((BATCH, NUM_ELEM, HIDDEN), jnp.float32),
        scratch_shapes=[
            pltpu.VMEM((K, C, NUM_ELEM, HIDDEN), jnp.float32),
            pltpu.VMEM((K, C, NUM_ELEM, HIDDEN), jnp.float32),
            pltpu.SemaphoreType.DMA((K,)),
            pltpu.SemaphoreType.DMA((K,)),
        ],
    )(batch_elem_emb, ids2, emb_table, gamma2, beta2)
